# baseline (speedup 1.0000x reference)
import functools

import jax
import jax.numpy as jnp
from jax import lax
from jax.experimental import pallas as pl
from jax.experimental.pallas import tpu as pltpu
from jax.experimental.pallas import tpu_sc as plsc

_NC = 2
_NS = 16
_NW = _NC * _NS
_RB = 8


def kernel(token_ids, W):
    B, S = token_ids.shape
    dim = W.shape[1]
    rows_per_w = B // _NW
    rounds = rows_per_w // _RB

    mesh = plsc.VectorSubcoreMesh(core_axis_name="c", subcore_axis_name="s")

    @functools.partial(
        pl.kernel,
        mesh=mesh,
        out_type=jax.ShapeDtypeStruct((B, S, dim), W.dtype),
        scratch_types=[
            pltpu.VMEM((rows_per_w, S), jnp.int32),
            pltpu.VMEM((_RB, S, dim), jnp.float32),
            pltpu.VMEM((_RB, S, dim), jnp.float32),
            pltpu.SemaphoreType.DMA,
            pltpu.SemaphoreType.DMA,
            pltpu.SemaphoreType.DMA,
            pltpu.SemaphoreType.DMA,
        ],
        compiler_params=pltpu.CompilerParams(use_tc_tiling_on_sc=False),
    )
    def gather_kernel(w_hbm, i_hbm, o_hbm, idx_v, buf0, buf1,
                      gsem0, gsem1, wsem0, wsem1):
        wid = lax.axis_index("s") * _NC + lax.axis_index("c")
        base = wid * rows_per_w
        pltpu.sync_copy(i_hbm.at[pl.ds(base, rows_per_w)], idx_v)

        bufs = (buf0, buf1)
        gsems = (gsem0, gsem1)
        wsems = (wsem0, wsem1)

        def fire(r, slot):
            for b in range(_RB):
                pltpu.async_copy(
                    w_hbm.at[idx_v.at[r * _RB + b]],
                    bufs[slot].at[b],
                    gsems[slot],
                )

        def drain_gathers(slot):
            pltpu.make_async_copy(
                o_hbm.at[pl.ds(0, _RB)], bufs[slot], gsems[slot]
            ).wait()

        def start_wb(r, slot):
            pltpu.async_copy(
                bufs[slot], o_hbm.at[pl.ds(base + r * _RB, _RB)], wsems[slot]
            )

        def drain_wb(slot):
            pltpu.make_async_copy(
                bufs[slot], o_hbm.at[pl.ds(base, _RB)], wsems[slot]
            ).wait()

        fire(0, 0)
        drain_gathers(0)
        fire(1, 1)
        start_wb(0, 0)
        drain_gathers(1)
        drain_wb(0)
        fire(2, 0)
        start_wb(1, 1)

        @pl.loop(2, rounds - 2, step=2)
        def _(j):
            drain_gathers(0)
            drain_wb(1)
            fire(j + 1, 1)
            start_wb(j, 0)
            drain_gathers(1)
            drain_wb(0)
            fire(j + 2, 0)
            start_wb(j + 1, 1)

        drain_gathers(0)
        drain_wb(1)
        fire(rounds - 1, 1)
        start_wb(rounds - 2, 0)
        drain_gathers(1)
        drain_wb(0)
        start_wb(rounds - 1, 1)
        drain_wb(1)

    return gather_kernel(W, token_ids)
